# baseline (speedup 1.0000x reference)
import jax
import jax.numpy as jnp
from jax.experimental import pallas as pl
from jax.experimental.pallas import tpu as pltpu

_G = 50


def _body(ctx_ref, v_ref, e_ref, w_ref, b_ref, o_ref, vacc, eacc):
    i = pl.program_id(0)

    @pl.when(i == 0)
    def _init():
        vacc[...] = jnp.zeros_like(vacc)
        eacc[...] = jnp.zeros_like(eacc)

    vacc[...] += jnp.sum(v_ref[...], axis=0, keepdims=True)
    eacc[...] += jnp.sum(e_ref[...], axis=0, keepdims=True)

    @pl.when(i == _G - 1)
    def _finish():
        n_v = v_ref.shape[0] * _G
        n_e = e_ref.shape[0] * _G
        v_agg = vacc[...] / n_v
        e_agg = eacc[...] / n_e
        d_ctx = ctx_ref.shape[1]
        d_feat = v_agg.shape[1]
        out = jnp.dot(ctx_ref[...], w_ref[0:d_ctx],
                      preferred_element_type=jnp.float32)
        out += jnp.dot(v_agg, w_ref[d_ctx:d_ctx + d_feat],
                       preferred_element_type=jnp.float32)
        out += jnp.dot(e_agg, w_ref[d_ctx + d_feat:],
                       preferred_element_type=jnp.float32)
        o_ref[...] = out + b_ref[...]


def kernel(context, vertex_data, edge_data, W, b):
    n_verts, d_feat = vertex_data.shape
    n_edges, d_edge = edge_data.shape
    d_ctx = context.shape[0]

    vc = n_verts // _G
    ec = n_edges // _G

    out = pl.pallas_call(
        _body,
        grid=(_G,),
        in_specs=[
            pl.BlockSpec((1, d_ctx), lambda i: (0, 0)),
            pl.BlockSpec((vc, d_feat), lambda i: (i, 0)),
            pl.BlockSpec((ec, d_edge), lambda i: (i, 0)),
            pl.BlockSpec((d_ctx + d_feat + d_edge, d_ctx), lambda i: (0, 0)),
            pl.BlockSpec((1, d_ctx), lambda i: (0, 0)),
        ],
        out_specs=pl.BlockSpec((1, d_ctx), lambda i: (0, 0)),
        out_shape=jax.ShapeDtypeStruct((1, d_ctx), jnp.float32),
        scratch_shapes=[
            pltpu.VMEM((1, d_feat), jnp.float32),
            pltpu.VMEM((1, d_edge), jnp.float32),
        ],
    )(context.reshape(1, d_ctx), vertex_data, edge_data, W,
      b.reshape(1, d_ctx))

    return out.reshape(d_ctx)


# speedup vs baseline: 1.0057x; 1.0057x over previous
import functools

import jax
import jax.numpy as jnp
from jax import lax
from jax.experimental import pallas as pl
from jax.experimental.pallas import tpu as pltpu
from jax.experimental.pallas import tpu_sc as plsc

_NC, _NS = 2, 16
_NW = _NC * _NS
_NBUF = 4
_NCHUNK = 10

_VS = 10
_VG = 5


def _edge_body(e_hbm, out_hbm, b0, b1, b2, b3, ov, s0, s1, s2, s3):
    wid = lax.axis_index("s") * _NC + lax.axis_index("c")
    bufs = (b0, b1, b2, b3)
    sems = (s0, s1, s2, s3)
    ch = b0.shape[0]
    per_w = ch * _NCHUNK
    base = wid * per_w

    def _copy(c, slot):
        return pltpu.make_async_copy(
            e_hbm.at[pl.ds(base + c * ch, ch)], bufs[slot], sems[slot])

    for c in range(_NBUF):
        _copy(c, c).start()

    accs = [jnp.zeros((16,), jnp.float32) for _ in range(4)]
    for c in range(_NCHUNK):
        slot = c % _NBUF
        _copy(c, slot).wait()
        buf = bufs[slot]

        def _rbody(i, a, buf=buf):
            r = i * 8
            return (a[0] + buf[r] + buf[r + 4],
                    a[1] + buf[r + 1] + buf[r + 5],
                    a[2] + buf[r + 2] + buf[r + 6],
                    a[3] + buf[r + 3] + buf[r + 7])

        accs = list(lax.fori_loop(0, ch // 8, _rbody, tuple(accs)))
        if c + _NBUF < _NCHUNK:
            _copy(c + _NBUF, slot).start()

    ov[...] = (accs[0] + accs[1]) + (accs[2] + accs[3])
    pltpu.sync_copy(ov, out_hbm.at[wid])


def _edge_partials(edge_data):
    n_edges = edge_data.shape[0]
    ch = n_edges // (_NW * _NCHUNK)
    kern = pl.kernel(
        _edge_body,
        out_type=jax.ShapeDtypeStruct((_NW, 16), jnp.float32),
        mesh=plsc.VectorSubcoreMesh(
            core_axis_name="c", subcore_axis_name="s",
            num_cores=_NC, num_subcores=_NS),
        scratch_types=(
            [pltpu.VMEM((ch, 16), jnp.float32) for _ in range(_NBUF)]
            + [pltpu.VMEM((16,), jnp.float32)]
            + [pltpu.SemaphoreType.DMA for _ in range(_NBUF)]),
        compiler_params=pltpu.CompilerParams(use_tc_tiling_on_sc=False),
    )
    return kern(edge_data)


def _vertex_body(ctx_ref, *rest):
    v_refs = rest[:_VS]
    w_ref, b_ref, o_ref, vacc = rest[_VS], rest[_VS + 1], rest[_VS + 2], rest[_VS + 3]
    i = pl.program_id(0)

    @pl.when(i == 0)
    def _init():
        vacc[...] = jnp.zeros_like(vacc)

    s = jnp.sum(v_refs[0][...], axis=0, keepdims=True)
    for vr in v_refs[1:]:
        s += jnp.sum(vr[...], axis=0, keepdims=True)
    vacc[...] += s

    @pl.when(i == _VG - 1)
    def _finish():
        d_ctx = ctx_ref.shape[1]
        d_feat = vacc.shape[1]
        n_v = v_refs[0].shape[0] * _VS * _VG
        out = jnp.dot(ctx_ref[...], w_ref[0:d_ctx],
                      preferred_element_type=jnp.float32)
        out += jnp.dot(vacc[...] / n_v, w_ref[d_ctx:d_ctx + d_feat],
                       preferred_element_type=jnp.float32)
        o_ref[...] = out + b_ref[...]


def _vertex_part(context, vertex_data, W, b):
    n_verts, d_feat = vertex_data.shape
    d_ctx = context.shape[0]
    d_tot = W.shape[0]
    vr = n_verts // (_VS * _VG)

    def _vmap(j):
        return lambda i, j=j: (_VG * j + i, 0)

    return pl.pallas_call(
        _vertex_body,
        grid=(_VG,),
        in_specs=(
            [pl.BlockSpec((1, d_ctx), lambda i: (0, 0))]
            + [pl.BlockSpec((vr, d_feat), _vmap(j)) for j in range(_VS)]
            + [pl.BlockSpec((d_tot, d_ctx), lambda i: (0, 0)),
               pl.BlockSpec((1, d_ctx), lambda i: (0, 0))]),
        out_specs=pl.BlockSpec((1, d_ctx), lambda i: (0, 0)),
        out_shape=jax.ShapeDtypeStruct((1, d_ctx), jnp.float32),
        scratch_shapes=[pltpu.VMEM((1, d_feat), jnp.float32)],
    )(context.reshape(1, d_ctx), *([vertex_data] * _VS), W,
      b.reshape(1, d_ctx))


def _combine_body(pa_ref, pb_ref, w_ref, o_ref, *, n_edges, d_we0):
    e_agg = jnp.sum(pa_ref[...], axis=0, keepdims=True) / n_edges
    d_edge = pa_ref.shape[1]
    o_ref[...] = pb_ref[...] + jnp.dot(
        e_agg, w_ref[d_we0:d_we0 + d_edge],
        preferred_element_type=jnp.float32)


def kernel(context, vertex_data, edge_data, W, b):
    n_edges, d_edge = edge_data.shape
    d_ctx = context.shape[0]
    d_feat = vertex_data.shape[1]
    d_tot = W.shape[0]

    pa = _edge_partials(edge_data)
    pb = _vertex_part(context, vertex_data, W, b)

    out = pl.pallas_call(
        functools.partial(_combine_body, n_edges=n_edges,
                          d_we0=d_ctx + d_feat),
        in_specs=[
            pl.BlockSpec((_NW, d_edge), lambda: (0, 0)),
            pl.BlockSpec((1, d_ctx), lambda: (0, 0)),
            pl.BlockSpec((d_tot, d_ctx), lambda: (0, 0)),
        ],
        out_specs=pl.BlockSpec((1, d_ctx), lambda: (0, 0)),
        out_shape=jax.ShapeDtypeStruct((1, d_ctx), jnp.float32),
    )(pa, pb, W)

    return out.reshape(d_ctx)


# speedup vs baseline: 4.8382x; 4.8110x over previous
import functools

import jax
import jax.numpy as jnp
from jax import lax
from jax.experimental import pallas as pl
from jax.experimental.pallas import tpu as pltpu
from jax.experimental.pallas import tpu_sc as plsc

_NC, _NS = 2, 16
_NW = _NC * _NS
_NBUF = 4
_ECH = 1280
_NROUND = 8

_VS = 10
_VG = 5


def _edge_body(e_hbm, out_hbm, b0, b1, b2, b3, ov, s0, s1, s2, s3):
    wid = lax.axis_index("s") * _NC + lax.axis_index("c")
    bufs = (b0, b1, b2, b3)
    sems = (s0, s1, s2, s3)
    n_edges = e_hbm.shape[1]
    nch = n_edges // _ECH

    def _copy(j, slot):
        cid = wid + j * _NW
        return pltpu.make_async_copy(
            e_hbm.at[:, pl.ds(cid * _ECH, _ECH)], bufs[slot], sems[slot])

    def _valid(j):
        return wid + j * _NW < nch

    for j in range(_NBUF):
        if (j + 1) * _NW <= nch:
            _copy(j, j).start()
        else:
            @pl.when(_valid(j))
            def _():
                _copy(j, j).start()

    def _chunk_acc(buf, accs, n=_ECH // 16):
        def body(i, a):
            base = i * 16
            return tuple(a[r] + buf[r, pl.ds(base, 16)] for r in range(16))
        return lax.fori_loop(0, n, body, accs)

    accs = tuple(jnp.zeros((16,), jnp.float32) for _ in range(16))
    for j in range(_NROUND):
        slot = j % _NBUF
        buf = bufs[slot]
        if (j + 1) * _NW <= nch:
            _copy(j, slot).wait()
            accs = _chunk_acc(buf, accs)
        else:
            @pl.when(_valid(j))
            def _():
                _copy(j, slot).wait()
            n_dyn = jnp.where(_valid(j), _ECH // 16, 0)
            accs = _chunk_acc(buf, accs, n=n_dyn)
        nxt = j + _NBUF
        if nxt < _NROUND:
            if (nxt + 1) * _NW <= nch:
                _copy(nxt, slot).start()
            else:
                @pl.when(_valid(nxt))
                def _():
                    _copy(nxt, slot).start()

    for r in range(16):
        ov[r] = accs[r]
    pltpu.sync_copy(ov, out_hbm.at[wid])


def _edge_partials(edge_t):
    kern = pl.kernel(
        _edge_body,
        out_type=jax.ShapeDtypeStruct((_NW, 16, 16), jnp.float32),
        mesh=plsc.VectorSubcoreMesh(
            core_axis_name="c", subcore_axis_name="s",
            num_cores=_NC, num_subcores=_NS),
        scratch_types=(
            [pltpu.VMEM((16, _ECH), jnp.float32) for _ in range(_NBUF)]
            + [pltpu.VMEM((16, 16), jnp.float32)]
            + [pltpu.SemaphoreType.DMA for _ in range(_NBUF)]),
    )
    return kern(edge_t)


def _vertex_body(ctx_ref, *rest):
    v_refs = rest[:_VS]
    w_ref, b_ref, o_ref, vacc = rest[_VS], rest[_VS + 1], rest[_VS + 2], rest[_VS + 3]
    i = pl.program_id(0)

    @pl.when(i == 0)
    def _init():
        vacc[...] = jnp.zeros_like(vacc)

    s = jnp.sum(v_refs[0][...], axis=0, keepdims=True)
    for vr in v_refs[1:]:
        s += jnp.sum(vr[...], axis=0, keepdims=True)
    vacc[...] += s

    @pl.when(i == _VG - 1)
    def _finish():
        d_ctx = ctx_ref.shape[1]
        d_feat = vacc.shape[1]
        n_v = v_refs[0].shape[0] * _VS * _VG
        out = jnp.dot(ctx_ref[...], w_ref[0:d_ctx],
                      preferred_element_type=jnp.float32)
        out += jnp.dot(vacc[...] / n_v, w_ref[d_ctx:d_ctx + d_feat],
                       preferred_element_type=jnp.float32)
        o_ref[...] = out + b_ref[...]


def _vertex_part(context, vertex_data, W, b):
    n_verts, d_feat = vertex_data.shape
    d_ctx = context.shape[0]
    d_tot = W.shape[0]
    vr = n_verts // (_VS * _VG)

    def _vmap(j):
        return lambda i, j=j: (_VG * j + i, 0)

    return pl.pallas_call(
        _vertex_body,
        grid=(_VG,),
        in_specs=(
            [pl.BlockSpec((1, d_ctx), lambda i: (0, 0))]
            + [pl.BlockSpec((vr, d_feat), _vmap(j)) for j in range(_VS)]
            + [pl.BlockSpec((d_tot, d_ctx), lambda i: (0, 0)),
               pl.BlockSpec((1, d_ctx), lambda i: (0, 0))]),
        out_specs=pl.BlockSpec((1, d_ctx), lambda i: (0, 0)),
        out_shape=jax.ShapeDtypeStruct((1, d_ctx), jnp.float32),
        scratch_shapes=[pltpu.VMEM((1, d_feat), jnp.float32)],
    )(context.reshape(1, d_ctx), *([vertex_data] * _VS), W,
      b.reshape(1, d_ctx))


def _combine_body(pa_ref, pb_ref, w_ref, o_ref, *, n_edges, d_we0):
    d_edge = pa_ref.shape[1]
    t = jnp.sum(pa_ref[...], axis=0)
    s = jnp.sum(t, axis=1, keepdims=True) / n_edges
    o_ref[...] = pb_ref[...] + lax.dot_general(
        s, w_ref[d_we0:d_we0 + d_edge],
        dimension_numbers=(((0,), (0,)), ((), ())),
        preferred_element_type=jnp.float32)


def kernel(context, vertex_data, edge_data, W, b):
    n_edges, d_edge = edge_data.shape
    d_ctx = context.shape[0]
    d_feat = vertex_data.shape[1]
    d_tot = W.shape[0]

    pa = _edge_partials(edge_data.T)
    pb = _vertex_part(context, vertex_data, W, b)

    out = pl.pallas_call(
        functools.partial(_combine_body, n_edges=n_edges,
                          d_we0=d_ctx + d_feat),
        in_specs=[
            pl.BlockSpec((_NW, d_edge, 16), lambda: (0, 0, 0)),
            pl.BlockSpec((1, d_ctx), lambda: (0, 0)),
            pl.BlockSpec((d_tot, d_ctx), lambda: (0, 0)),
        ],
        out_specs=pl.BlockSpec((1, d_ctx), lambda: (0, 0)),
        out_shape=jax.ShapeDtypeStruct((1, d_ctx), jnp.float32),
    )(pa, pb, W)

    return out.reshape(d_ctx)


# speedup vs baseline: 12.9183x; 2.6701x over previous
import functools

import jax
import jax.numpy as jnp
from jax import lax
from jax.experimental import pallas as pl
from jax.experimental.pallas import tpu as pltpu

_G = 5
_SV = 25
_SE = 10


def _body(*refs):
    ctx_ref = refs[0]
    v_refs = refs[1:1 + _SV]
    e_refs = refs[1 + _SV:1 + _SV + _SE]
    w_ref, b_ref, o_ref, vacc, eacc = refs[1 + _SV + _SE:]
    i = pl.program_id(0)

    @pl.when(i == 0)
    def _init():
        vacc[...] = jnp.zeros_like(vacc)
        eacc[...] = jnp.zeros_like(eacc)

    s = jnp.sum(v_refs[0][...], axis=0, keepdims=True)
    for vr in v_refs[1:]:
        s += jnp.sum(vr[...], axis=0, keepdims=True)
    vacc[...] += s

    d_edge = e_refs[0].shape[0]
    ec = e_refs[0].shape[1]
    t = e_refs[0][...].reshape(d_edge, ec // 128, 128).sum(axis=1)
    for er in e_refs[1:]:
        t += er[...].reshape(d_edge, ec // 128, 128).sum(axis=1)
    eacc[...] += t

    @pl.when(i == _G - 1)
    def _finish():
        d_ctx = ctx_ref.shape[1]
        d_feat = vacc.shape[1]
        n_v = v_refs[0].shape[0] * _SV * _G
        n_e = ec * _SE * _G
        out = jnp.dot(ctx_ref[...], w_ref[0:d_ctx],
                      preferred_element_type=jnp.float32)
        out += jnp.dot(vacc[...] / n_v, w_ref[d_ctx:d_ctx + d_feat],
                       preferred_element_type=jnp.float32)
        e_sum = jnp.sum(eacc[...], axis=1, keepdims=True) / n_e
        out += lax.dot_general(
            e_sum, w_ref[d_ctx + d_feat:d_ctx + d_feat + d_edge],
            dimension_numbers=(((0,), (0,)), ((), ())),
            preferred_element_type=jnp.float32)
        o_ref[...] = out + b_ref[...]


def kernel(context, vertex_data, edge_data, W, b):
    n_verts, d_feat = vertex_data.shape
    n_edges, d_edge = edge_data.shape
    d_ctx = context.shape[0]
    d_tot = W.shape[0]

    edge_t = edge_data.T
    vc = n_verts // (_SV * _G)
    ec = n_edges // (_SE * _G)

    def _vmap(j):
        return lambda i, j=j: (_G * j + i, 0)

    def _emap(j):
        return lambda i, j=j: (0, _G * j + i)

    out = pl.pallas_call(
        _body,
        grid=(_G,),
        in_specs=(
            [pl.BlockSpec((1, d_ctx), lambda i: (0, 0))]
            + [pl.BlockSpec((vc, d_feat), _vmap(j)) for j in range(_SV)]
            + [pl.BlockSpec((d_edge, ec), _emap(j)) for j in range(_SE)]
            + [pl.BlockSpec((d_tot, d_ctx), lambda i: (0, 0)),
               pl.BlockSpec((1, d_ctx), lambda i: (0, 0))]),
        out_specs=pl.BlockSpec((1, d_ctx), lambda i: (0, 0)),
        out_shape=jax.ShapeDtypeStruct((1, d_ctx), jnp.float32),
        scratch_shapes=[pltpu.VMEM((1, d_feat), jnp.float32),
                        pltpu.VMEM((d_edge, 128), jnp.float32)],
    )(context.reshape(1, d_ctx), *([vertex_data] * _SV),
      *([edge_t] * _SE), W, b.reshape(1, d_ctx))

    return out.reshape(d_ctx)
